# baseline (speedup 1.0000x reference)
import functools

import jax
import jax.numpy as jnp
from jax import lax
from jax.experimental import pallas as pl
from jax.experimental.pallas import tpu as pltpu
from jax.experimental.pallas import tpu_sc as plsc

_BETA = 0.25
_BM = 512
_BK = 8192


def _make_argmin_body(use_esq):
    def body(z_ref, emb_ref, zsq_ref, esq_ref, idx_ref, dsum_ref):
        zb2 = z_ref[...] * (-2.0)
        zsq = zsq_ref[...]
        k_total = emb_ref.shape[0]
        lanes = lax.broadcasted_iota(
            jnp.int32, (_BM, 128), 1).astype(jnp.float32)
        nsl = _BK // 128
        run_min = jnp.full((_BM, 1), jnp.inf, dtype=jnp.float32)
        run_idx = jnp.zeros((_BM, 1), dtype=jnp.float32)
        for j in range(k_total // _BK):
            eb = emb_ref[j * _BK:(j + 1) * _BK, :]
            mmneg = lax.dot_general(zb2, eb, (((1,), (1,)), ((), ())),
                                    preferred_element_type=jnp.float32)
            if use_esq:
                esq = esq_ref[:, j * _BK:(j + 1) * _BK]
                mmneg = (zsq + esq) + mmneg
            bv = bt = None
            for t in range(nsl):
                v = mmneg[:, t * 128:(t + 1) * 128]
                if not use_esq:
                    v = zsq + v
                if t == 0:
                    bv = v
                    bt = jnp.zeros((_BM, 128), dtype=jnp.float32)
                else:
                    lt = v < bv
                    bv = jnp.where(lt, v, bv)
                    bt = jnp.where(lt, jnp.float32(t * 128), bt)
            rowmin = jnp.min(bv, axis=1, keepdims=True)
            local = jnp.min(jnp.where(bv == rowmin, bt + lanes, float(_BK)),
                            axis=1, keepdims=True)
            better = rowmin < run_min
            run_min = jnp.where(better, rowmin, run_min)
            run_idx = jnp.where(better, local + float(j * _BK), run_idx)
        idx_ref[...] = run_idx.astype(jnp.int32)[None]
        dsum_ref[...] = jnp.sum(run_min).reshape(1, 1, 1)

    return body


def _argmin_call(z_flat, emb, zsq, esq_row, use_esq):
    m, d = z_flat.shape
    k = emb.shape[0]
    nblocks = m // _BM
    return pl.pallas_call(
        _make_argmin_body(use_esq),
        grid=(nblocks,),
        in_specs=[
            pl.BlockSpec((_BM, d), lambda i: (i, 0)),
            pl.BlockSpec((k, d), lambda i: (0, 0)),
            pl.BlockSpec((_BM, 1), lambda i: (i, 0)),
            pl.BlockSpec((1, k), lambda i: (0, 0)),
        ],
        out_specs=[
            pl.BlockSpec((1, _BM, 1), lambda i: (i, 0, 0)),
            pl.BlockSpec((1, 1, 1), lambda i: (i, 0, 0)),
        ],
        out_shape=[
            jax.ShapeDtypeStruct((nblocks, _BM, 1), jnp.int32),
            jax.ShapeDtypeStruct((nblocks, 1, 1), jnp.float32),
        ],
    )(z_flat, emb, zsq, esq_row)


def _sc_gather(table, idx2d):
    k, d = table.shape
    b = idx2d.shape[0] * 128
    nw = 32
    bpw = b // nw
    rpw = bpw // 128
    mesh = plsc.VectorSubcoreMesh(core_axis_name="c", subcore_axis_name="s")

    @functools.partial(
        pl.kernel,
        mesh=mesh,
        out_type=jax.ShapeDtypeStruct((b, d), jnp.float32),
        scratch_types=[
            pltpu.VMEM((rpw, 128), jnp.int32),
            pltpu.VMEM((bpw, d), jnp.float32),
            pltpu.SemaphoreType.DMA,
        ],
    )
    def gather_kernel(table_hbm, idx_hbm, out_hbm, idx_v, rows_v, sem):
        wid = lax.axis_index("s") * 2 + lax.axis_index("c")
        pltpu.sync_copy(idx_hbm.at[pl.ds(wid * rpw, rpw)], idx_v)
        copies = [
            pltpu.async_copy(table_hbm.at[idx_v.at[r]],
                             rows_v.at[pl.ds(r * 128, 128)], sem)
            for r in range(rpw)
        ]
        for c in copies:
            c.wait()
        pltpu.sync_copy(rows_v, out_hbm.at[pl.ds(wid * bpw, bpw)])

    return gather_kernel(table, idx2d)


def kernel(z, emb):
    b, d, h, w = z.shape
    m = b * h * w
    k = emb.shape[0]
    z_flat = jnp.transpose(z, (0, 2, 3, 1)).reshape(-1, d)
    zsq = jnp.sum(z_flat ** 2, axis=1, keepdims=True)

    def _fast(ops):
        zf, e, zs = ops
        dummy = jnp.zeros((1, k), jnp.float32)
        return _argmin_call(zf, e, zs, dummy, use_esq=False)

    def _exact(ops):
        zf, e, zs = ops
        esq = jnp.sum(e ** 2, axis=1)
        return _argmin_call(zf, e, zs, esq.reshape(1, k), use_esq=True)

    mh = m // 2
    pred = jnp.all(zsq >= 128.0)
    parts = []
    for lo in (0, mh):
        parts.append(lax.cond(pred, _fast, _exact,
                              (z_flat[lo:lo + mh], emb, zsq[lo:lo + mh])))
    idx = jnp.concatenate([p[0].reshape(mh) for p in parts])
    zq_parts = [_sc_gather(emb, p[0].reshape(mh // 128, 128))
                for p in parts]
    zq_flat = jnp.concatenate(zq_parts)
    z_q = jnp.transpose(zq_flat.reshape(b, h, w, d), (0, 3, 1, 2))
    dsum = parts[0][1].sum() + parts[1][1].sum()
    loss = (1.0 + _BETA) * (dsum / (m * d))
    return z_q, loss, idx


# speedup vs baseline: 1.3059x; 1.3059x over previous
import functools

import jax
import jax.numpy as jnp
from jax import lax
from jax.experimental import pallas as pl
from jax.experimental.pallas import tpu as pltpu
from jax.experimental.pallas import tpu_sc as plsc

_BETA = 0.25
_BM = 512
_BK = 8192


def _make_argmin_body(use_esq):
    def body_fast(z_ref, emb_ref, idx_ref, dsum_ref, zmin_ref):
        return _body(z_ref, emb_ref, None, idx_ref, dsum_ref, zmin_ref)

    def body_exact(z_ref, emb_ref, zsq_ref, esq_ref, idx_ref, dsum_ref):
        return _body(z_ref, emb_ref, (zsq_ref, esq_ref), idx_ref, dsum_ref,
                     None)

    def _body(z_ref, emb_ref, norm_refs, idx_ref, dsum_ref, zmin_ref):
        zb = z_ref[...]
        zb2 = zb * (-2.0)
        if use_esq:
            zsq_ref, esq_ref = norm_refs
            zsq = zsq_ref[...]
        else:
            zsq = jnp.sum(zb ** 2, axis=1, keepdims=True)
        k_total = emb_ref.shape[0]
        lanes = lax.broadcasted_iota(
            jnp.int32, (_BM, 128), 1).astype(jnp.float32)
        nsl = _BK // 128
        run_min = jnp.full((_BM, 1), jnp.inf, dtype=jnp.float32)
        run_idx = jnp.zeros((_BM, 1), dtype=jnp.float32)
        for j in range(k_total // _BK):
            eb = emb_ref[j * _BK:(j + 1) * _BK, :]
            mmneg = lax.dot_general(zb2, eb, (((1,), (1,)), ((), ())),
                                    preferred_element_type=jnp.float32)
            if use_esq:
                esq = esq_ref[:, j * _BK:(j + 1) * _BK]
                mmneg = (zsq + esq) + mmneg
            bv = bt = None
            for t in range(nsl):
                v = mmneg[:, t * 128:(t + 1) * 128]
                if not use_esq:
                    v = zsq + v
                if t == 0:
                    bv = v
                    bt = jnp.zeros((_BM, 128), dtype=jnp.float32)
                else:
                    lt = v < bv
                    bv = jnp.where(lt, v, bv)
                    bt = jnp.where(lt, jnp.float32(t * 128), bt)
            rowmin = jnp.min(bv, axis=1, keepdims=True)
            local = jnp.min(jnp.where(bv == rowmin, bt + lanes, float(_BK)),
                            axis=1, keepdims=True)
            better = rowmin < run_min
            run_min = jnp.where(better, rowmin, run_min)
            run_idx = jnp.where(better, local + float(j * _BK), run_idx)
        idx_ref[...] = run_idx.astype(jnp.int32)[None]
        dsum_ref[...] = jnp.sum(run_min).reshape(1, 1, 1)
        if not use_esq:
            zmin_ref[...] = jnp.min(zsq).reshape(1, 1, 1)

    return body_exact if use_esq else body_fast


def _argmin_call_fast(z_flat, emb):
    m, d = z_flat.shape
    k = emb.shape[0]
    nblocks = m // _BM
    return pl.pallas_call(
        _make_argmin_body(False),
        grid=(nblocks,),
        in_specs=[
            pl.BlockSpec((_BM, d), lambda i: (i, 0)),
            pl.BlockSpec((k, d), lambda i: (0, 0)),
        ],
        out_specs=[
            pl.BlockSpec((1, _BM, 1), lambda i: (i, 0, 0)),
            pl.BlockSpec((1, 1, 1), lambda i: (i, 0, 0)),
            pl.BlockSpec((1, 1, 1), lambda i: (i, 0, 0)),
        ],
        out_shape=[
            jax.ShapeDtypeStruct((nblocks, _BM, 1), jnp.int32),
            jax.ShapeDtypeStruct((nblocks, 1, 1), jnp.float32),
            jax.ShapeDtypeStruct((nblocks, 1, 1), jnp.float32),
        ],
    )(z_flat, emb)


def _argmin_call_exact(z_flat, emb, zsq, esq_row):
    m, d = z_flat.shape
    k = emb.shape[0]
    nblocks = m // _BM
    return pl.pallas_call(
        _make_argmin_body(True),
        grid=(nblocks,),
        in_specs=[
            pl.BlockSpec((_BM, d), lambda i: (i, 0)),
            pl.BlockSpec((k, d), lambda i: (0, 0)),
            pl.BlockSpec((_BM, 1), lambda i: (i, 0)),
            pl.BlockSpec((1, k), lambda i: (0, 0)),
        ],
        out_specs=[
            pl.BlockSpec((1, _BM, 1), lambda i: (i, 0, 0)),
            pl.BlockSpec((1, 1, 1), lambda i: (i, 0, 0)),
        ],
        out_shape=[
            jax.ShapeDtypeStruct((nblocks, _BM, 1), jnp.int32),
            jax.ShapeDtypeStruct((nblocks, 1, 1), jnp.float32),
        ],
    )(z_flat, emb, zsq, esq_row)


def _sc_gather(table, idx2d):
    k, d = table.shape
    b = idx2d.shape[0] * 128
    nw = 32
    bpw = b // nw
    rpw = bpw // 128
    mesh = plsc.VectorSubcoreMesh(core_axis_name="c", subcore_axis_name="s")

    @functools.partial(
        pl.kernel,
        mesh=mesh,
        out_type=jax.ShapeDtypeStruct((b, d), jnp.float32),
        scratch_types=[
            pltpu.VMEM((rpw, 128), jnp.int32),
            pltpu.VMEM((bpw, d), jnp.float32),
            pltpu.SemaphoreType.DMA,
        ],
    )
    def gather_kernel(table_hbm, idx_hbm, out_hbm, idx_v, rows_v, sem):
        wid = lax.axis_index("s") * 2 + lax.axis_index("c")
        pltpu.sync_copy(idx_hbm.at[pl.ds(wid * rpw, rpw)], idx_v)
        copies = [
            pltpu.async_copy(table_hbm.at[idx_v.at[r]],
                             rows_v.at[pl.ds(r * 128, 128)], sem)
            for r in range(rpw)
        ]
        for c in copies:
            c.wait()
        pltpu.sync_copy(rows_v, out_hbm.at[pl.ds(wid * bpw, bpw)])

    return gather_kernel(table, idx2d)


def kernel(z, emb):
    b, d, h, w = z.shape
    m = b * h * w
    k = emb.shape[0]
    z_flat = jnp.transpose(z, (0, 2, 3, 1)).reshape(-1, d)
    idx3_f, dsum_f, zmin_f = _argmin_call_fast(z_flat, emb)

    def _ok(ops):
        return idx3_f, dsum_f

    def _exact(ops):
        zf, e = ops
        zsq = jnp.sum(zf ** 2, axis=1, keepdims=True)
        esq = jnp.sum(e ** 2, axis=1)
        return _argmin_call_exact(zf, e, zsq, esq.reshape(1, k))

    idx3, dsum = lax.cond(jnp.all(zmin_f >= 128.0), _ok, _exact,
                          (z_flat, emb))
    idx = idx3.reshape(m)
    zq_flat = _sc_gather(emb, idx.reshape(m // 128, 128))
    z_q = jnp.transpose(zq_flat.reshape(b, h, w, d), (0, 3, 1, 2))
    loss = (1.0 + _BETA) * (jnp.sum(dsum) / (m * d))
    return z_q, loss, idx
